# Initial kernel scaffold; baseline (speedup 1.0000x reference)
import jax
import jax.numpy as jnp
from jax.experimental import pallas as pl


def kernel(x, edge_index, edge_weight, W0, b0, W1, b1, Wp, bp):
    raise NotImplementedError("write your pallas kernel here")


# speedup vs baseline: 4.3000x; 4.3000x over previous
import functools

import jax
import jax.numpy as jnp
from jax import lax
from jax.experimental import pallas as pl
from jax.experimental.pallas import tpu as pltpu
from jax.experimental.pallas import tpu_sc as plsc

NC = 2
NS = 16
NW = NC * NS
LANES = 16
C = 128


def _make_agg(npad, d, nch):
  rows_per_tile = npad // NS
  full, rem = divmod(rows_per_tile, C)
  mesh = plsc.VectorSubcoreMesh(core_axis_name="c", subcore_axis_name="s")

  @functools.partial(
      pl.kernel,
      out_type=jax.ShapeDtypeStruct((NC, npad, d), jnp.float32),
      mesh=mesh,
      scratch_types=[
          pltpu.VMEM((nch, C), jnp.int32),
          pltpu.VMEM((nch, C), jnp.int32),
          pltpu.VMEM((nch, C), jnp.float32),
          pltpu.VMEM((C, d), jnp.float32),
          pltpu.VMEM_SHARED((npad, d), jnp.float32),
          pltpu.SemaphoreType.DMA,
      ],
  )
  def agg(table_hbm, src_hbm, dst_hbm, w_hbm, out_hbm,
          src_v, dst_v, w_v, rowbuf, acc, gsem):
    c = lax.axis_index("c")
    s = lax.axis_index("s")
    wid = s * NC + c

    pltpu.sync_copy(src_hbm.at[wid], src_v)
    pltpu.sync_copy(dst_hbm.at[wid], dst_v)
    pltpu.sync_copy(w_hbm.at[wid], w_v)

    zeros16 = jnp.zeros((LANES,), jnp.float32)

    def zrow(i, carry):
      for j in range(d // LANES):
        rowbuf[i, pl.ds(j * LANES, LANES)] = zeros16
      return carry

    lax.fori_loop(0, C, zrow, 0)
    row0 = s * rows_per_tile
    for kk in range(full):
      pltpu.sync_copy(rowbuf, acc.at[pl.ds(row0 + kk * C, C)])
    if rem:
      pltpu.sync_copy(rowbuf.at[pl.ds(0, rem)],
                      acc.at[pl.ds(row0 + full * C, rem)])
    plsc.subcore_barrier()

    def chunk(k, carry):
      pltpu.async_copy(table_hbm.at[src_v.at[k]], rowbuf, gsem).wait()

      def scale_group(g, carry2):
        wv = w_v[k, pl.ds(g * LANES, LANES)]
        for eu in range(LANES):
          e = g * LANES + eu
          wb = jnp.full((LANES,), wv[eu], jnp.float32)
          for j in range(d // LANES):
            sl = pl.ds(j * LANES, LANES)
            rowbuf[e, sl] = rowbuf[e, sl] * wb
        return carry2

      lax.fori_loop(0, C // LANES, scale_group, 0)
      pltpu.sync_copy(rowbuf, acc.at[dst_v.at[k]], add=True)
      return carry

    lax.fori_loop(0, nch, chunk, 0)
    plsc.subcore_barrier()

    pltpu.sync_copy(acc.at[pl.ds(row0, rows_per_tile)],
                    out_hbm.at[c, pl.ds(row0, rows_per_tile)])

  return agg


def _mm_relu(p, w, b):
  _, n, d = p.shape
  h = w.shape[1]
  bn = 1264

  def body(p_ref, w_ref, b_ref, o_ref):
    ps = p_ref[0] + p_ref[1]
    o_ref[...] = jnp.maximum(
        jnp.dot(ps, w_ref[...], preferred_element_type=jnp.float32)
        + b_ref[...], 0.0)

  return pl.pallas_call(
      body,
      grid=(n // bn,),
      in_specs=[
          pl.BlockSpec((2, bn, d), lambda i: (0, i, 0)),
          pl.BlockSpec((d, h), lambda i: (0, 0)),
          pl.BlockSpec((1, h), lambda i: (0, 0)),
      ],
      out_specs=pl.BlockSpec((bn, h), lambda i: (i, 0)),
      out_shape=jax.ShapeDtypeStruct((n, h), jnp.float32),
  )(p, w, b.reshape(1, -1))


def _readout(hfin, wp, bp, n):
  d = hfin.shape[1]
  outd = wp.shape[1]
  bn = 1000
  nb = n // bn

  def body(h_ref, wpt_ref, wpb_ref, bp_ref, o_ref):
    i = pl.program_id(0)

    @pl.when(i == 0)
    def _():
      o_ref[...] = jnp.zeros_like(o_ref)
      o_ref[0:1, :] = bp_ref[...]

    hb = h_ref[...]
    m = jnp.max(hb, axis=1, keepdims=True)
    sm = jnp.sum(hb, axis=1, keepdims=True)
    dn = (((0,), (0,)), ((), ()))
    contrib = (lax.dot_general(m, wpt_ref[...], dn,
                               preferred_element_type=jnp.float32)
               + lax.dot_general(sm, wpb_ref[...], dn,
                                 preferred_element_type=jnp.float32))
    o_ref[0:1, :] += contrib

  out = pl.pallas_call(
      body,
      grid=(nb,),
      in_specs=[
          pl.BlockSpec((bn, d), lambda i: (i, 0)),
          pl.BlockSpec((bn, outd), lambda i: (i, 0)),
          pl.BlockSpec((bn, outd), lambda i: (nb + i, 0)),
          pl.BlockSpec((1, outd), lambda i: (0, 0)),
      ],
      out_specs=pl.BlockSpec((8, outd), lambda i: (0, 0)),
      out_shape=jax.ShapeDtypeStruct((8, outd), jnp.float32),
  )(hfin, wp, wp, bp.reshape(1, -1))
  return out[0]


def kernel(x, edge_index, edge_weight, W0, b0, W1, b1, Wp, bp):
  n, d = x.shape
  e = edge_index.shape[1]
  nch = -(-e // (NW * C))
  epad = NW * nch * C
  npad = -(-n // (NS * 8)) * (NS * 8)

  src = edge_index[0]
  dst = edge_index[1]
  pad = epad - e
  srcp = jnp.concatenate([src, jnp.zeros((pad,), jnp.int32)]).reshape(NW, nch, C)
  dstp = jnp.concatenate([dst, jnp.zeros((pad,), jnp.int32)]).reshape(NW, nch, C)
  wp_e = jnp.concatenate(
      [edge_weight, jnp.zeros((pad,), jnp.float32)]).reshape(NW, nch, C)

  agg = _make_agg(npad, d, nch)
  p0 = agg(x, srcp, dstp, wp_e)
  h1 = _mm_relu(p0, W0, b0)
  p1 = agg(h1, srcp, dstp, wp_e)
  h2 = _mm_relu(p1, W1, b1)
  return _readout(h2, Wp, bp, n)
